# Initial kernel scaffold; baseline (speedup 1.0000x reference)
import jax
import jax.numpy as jnp
from jax.experimental import pallas as pl


def kernel(sentences_idx, table):
    raise NotImplementedError("write your pallas kernel here")


# speedup vs baseline: 1.0992x; 1.0992x over previous
import functools

import jax
import jax.numpy as jnp
from jax import lax
from jax.experimental import pallas as pl
from jax.experimental.pallas import tpu as pltpu
from jax.experimental.pallas import tpu_sc as plsc


def _gather_rows(idx_flat, table, n_total, d):
    info = plsc.get_sparse_core_info()
    nw = info.num_cores * info.num_subcores
    per_w = n_total // nw
    chunk = 1280
    n_chunks = per_w // chunk

    mesh = plsc.VectorSubcoreMesh(core_axis_name="c", subcore_axis_name="s")

    @functools.partial(
        pl.kernel,
        mesh=mesh,
        out_type=jax.ShapeDtypeStruct((n_total, d), jnp.float32),
        scratch_types=[
            pltpu.VMEM((chunk,), jnp.int32),
            pltpu.VMEM((chunk, d), jnp.float32),
            pltpu.SemaphoreType.DMA,
        ],
        compiler_params=pltpu.CompilerParams(use_tc_tiling_on_sc=False),
    )
    def k(idx_hbm, tab_hbm, out_hbm, idx_v, rows_v, sem):
        wid = lax.axis_index("s") * info.num_cores + lax.axis_index("c")
        base = wid * per_w

        def body(c, carry):
            off = base + c * chunk
            pltpu.sync_copy(idx_hbm.at[pl.ds(off, chunk)], idx_v)
            pltpu.async_copy(tab_hbm.at[idx_v], rows_v, sem).wait()
            pltpu.sync_copy(rows_v, out_hbm.at[pl.ds(off, chunk)])
            return carry

        lax.fori_loop(0, n_chunks, body, 0)

    return k(idx_flat, table)


def kernel(sentences_idx, table):
    b, h = sentences_idx.shape
    v, d = table.shape
    n_total = b * h
    idx_flat = sentences_idx.reshape(n_total).astype(jnp.int32)
    out = _gather_rows(idx_flat, table, n_total, d)
    return out.reshape(b, h, d)


# speedup vs baseline: 1.1101x; 1.0099x over previous
import functools

import jax
import jax.numpy as jnp
from jax import lax
from jax.experimental import pallas as pl
from jax.experimental.pallas import tpu as pltpu
from jax.experimental.pallas import tpu_sc as plsc

_CHUNK = 1280


def _gather_rows(idx_flat, table, n_total, d):
    info = plsc.get_sparse_core_info()
    nw = info.num_cores * info.num_subcores
    per_w = n_total // nw
    chunk = _CHUNK
    n_chunks = per_w // chunk
    n_groups = n_chunks // 2

    mesh = plsc.VectorSubcoreMesh(core_axis_name="c", subcore_axis_name="s")

    @functools.partial(
        pl.kernel,
        mesh=mesh,
        out_type=jax.ShapeDtypeStruct((n_total, d), jnp.float32),
        scratch_types=[
            pltpu.VMEM((per_w,), jnp.int32),
            pltpu.VMEM((chunk, d), jnp.float32),
            pltpu.VMEM((chunk, d), jnp.float32),
            pltpu.SemaphoreType.DMA,
            pltpu.SemaphoreType.DMA,
        ],
        compiler_params=pltpu.CompilerParams(use_tc_tiling_on_sc=False),
    )
    def k(idx_hbm, tab_hbm, out_hbm, idx_v, r0, r1, s0, s1):
        wid = lax.axis_index("s") * info.num_cores + lax.axis_index("c")
        base = wid * per_w
        rows = (r0, r1)
        sems = (s0, s1)

        def gather(c, b):
            pltpu.async_copy(
                tab_hbm.at[idx_v.at[pl.ds(c * chunk, chunk)]], rows[b], sems[b])

        def store(c, b):
            pltpu.async_copy(
                rows[b], out_hbm.at[pl.ds(base + c * chunk, chunk)], sems[b])

        def wait(b):
            pltpu.make_async_copy(
                out_hbm.at[pl.ds(0, chunk)], rows[b], sems[b]).wait()

        pltpu.sync_copy(idx_hbm.at[pl.ds(base, per_w)], idx_v)

        gather(0, 0)
        wait(0)
        gather(1, 1)
        store(0, 0)
        wait(1)
        wait(0)
        gather(2, 0)
        store(1, 1)

        def body(i, carry):
            c0 = i * 2
            wait(0)
            wait(1)
            gather(c0 + 1, 1)
            store(c0, 0)
            wait(1)
            wait(0)
            gather(c0 + 2, 0)
            store(c0 + 1, 1)
            return carry

        lax.fori_loop(1, n_groups - 1, body, 0)

        cl = n_chunks - 2
        wait(0)
        wait(1)
        gather(cl + 1, 1)
        store(cl, 0)
        wait(1)
        store(cl + 1, 1)
        wait(0)
        wait(1)

    return k(idx_flat, table)


def kernel(sentences_idx, table):
    b, h = sentences_idx.shape
    v, d = table.shape
    n_total = b * h
    idx_flat = sentences_idx.reshape(n_total).astype(jnp.int32)
    out = _gather_rows(idx_flat, table, n_total, d)
    return out.reshape(b, h, d)


# speedup vs baseline: 1.1694x; 1.0534x over previous
import functools

import jax
import jax.numpy as jnp
from jax import lax
from jax.experimental import pallas as pl
from jax.experimental.pallas import tpu as pltpu
from jax.experimental.pallas import tpu_sc as plsc

_V = 1000000
_D = 32
_B = 16384
_H = 50
_NW = 32
_VBLK = 128
_NFULL = _V // _VBLK
_TAIL = _V - _NFULL * _VBLK
_MAIN = 7808
_PER_W = _MAIN // _NW


def _wid():
    return lax.axis_index("s") * 2 + lax.axis_index("c")


def _detile(tabT, idxT, tail):
    mesh = plsc.VectorSubcoreMesh(core_axis_name="c", subcore_axis_name="s")

    @functools.partial(
        pl.kernel,
        mesh=mesh,
        out_type=(
            jax.ShapeDtypeStruct((_V * _D,), jnp.float32),
            jax.ShapeDtypeStruct((_H * _B,), jnp.int32),
        ),
        scratch_types=[
            pltpu.VMEM((_D, _VBLK), jnp.float32),
            pltpu.VMEM((_D, _VBLK), jnp.float32),
            pltpu.VMEM((_VBLK * _D,), jnp.float32),
            pltpu.VMEM((_VBLK * _D,), jnp.float32),
            pltpu.VMEM((2, 2048), jnp.int32),
            pltpu.VMEM((_TAIL, _D), jnp.float32),
            pltpu.VMEM((_TAIL * _D,), jnp.float32),
            pltpu.SemaphoreType.DMA,
            pltpu.SemaphoreType.DMA,
            pltpu.SemaphoreType.DMA,
            pltpu.SemaphoreType.DMA,
        ],
        compiler_params=pltpu.CompilerParams(needs_layout_passes=False),
    )
    def k(tabT_hbm, idxT_hbm, tail_hbm, tab_lin, idx_lin,
          g0, g1, t0, t1, ibuf, gt, tt, sg0, sg1, st0, st1):
        i16 = lax.iota(jnp.int32, 16)
        w = _wid()
        gbuf = (g0, g1)
        tbuf = (t0, t1)
        sg = (sg0, sg1)
        st = (st0, st1)

        def blk_of(i):
            return w * _PER_W + i

        def gather_in(i, b):
            pltpu.async_copy(
                tabT_hbm.at[:, pl.ds(blk_of(i) * _VBLK, _VBLK)], gbuf[b], sg[b])

        def store_out(i, b):
            pltpu.async_copy(
                tbuf[b], tab_lin.at[pl.ds(blk_of(i) * _VBLK * _D, _VBLK * _D)],
                st[b])

        def wait_g(b):
            pltpu.make_async_copy(
                tabT_hbm.at[:, pl.ds(0, _VBLK)], gbuf[b], sg[b]).wait()

        def wait_s(b):
            pltpu.make_async_copy(
                tab_lin.at[pl.ds(0, _VBLK * _D)], tbuf[b], st[b]).wait()

        def transpose(b):
            for l in range(_VBLK):
                for kk in range(2):
                    vec = plsc.load_gather(
                        gbuf[b], [i16 + (kk * 16), jnp.full((16,), l, jnp.int32)])
                    tbuf[b][pl.ds(l * _D + kk * 16, 16)] = vec

        gather_in(0, 0)
        wait_g(0)
        gather_in(1, 1)
        transpose(0)
        store_out(0, 0)
        wait_g(1)
        gather_in(2, 0)
        transpose(1)
        store_out(1, 1)

        def body(p, carry):
            c0 = p * 2
            wait_g(0)
            wait_s(1)
            gather_in(c0 + 1, 1)
            transpose(0)
            store_out(c0, 0)
            wait_g(1)
            wait_s(0)
            gather_in(c0 + 2, 0)
            transpose(1)
            store_out(c0 + 1, 1)
            return carry

        lax.fori_loop(1, _PER_W // 2 - 1, body, 0)

        cl = _PER_W - 2
        wait_g(0)
        wait_s(1)
        gather_in(cl + 1, 1)
        transpose(0)
        store_out(cl, 0)
        wait_g(1)
        transpose(1)
        store_out(cl + 1, 1)
        wait_s(0)
        wait_s(1)

        @pl.when(w >= 28)
        def _extra():
            blk = _MAIN + (w - 28)
            pltpu.async_copy(
                tabT_hbm.at[:, pl.ds(blk * _VBLK, _VBLK)], g0, sg0)
            wait_g(0)
            transpose(0)
            pltpu.async_copy(
                t0, tab_lin.at[pl.ds(blk * _VBLK * _D, _VBLK * _D)], st0)
            wait_s(0)

        @pl.when(w == 27)
        def _tail():
            pltpu.sync_copy(tail_hbm, gt)
            for r in range(_TAIL):
                for kk in range(2):
                    tt[pl.ds(r * _D + kk * 16, 16)] = gt[r, pl.ds(kk * 16, 16)]
            pltpu.async_copy(
                tt, tab_lin.at[pl.ds(_NFULL * _VBLK * _D, _TAIL * _D)], st0)
            pltpu.make_async_copy(tab_lin.at[pl.ds(0, _TAIL * _D)], tt,
                                  st0).wait()

        def ichunk(q, carry):
            rp = q // 8
            cc = q % 8
            pltpu.sync_copy(
                idxT_hbm.at[pl.ds(rp * 2, 2), pl.ds(cc * 2048, 2048)], ibuf)
            pltpu.sync_copy(
                ibuf.at[0],
                idx_lin.at[pl.ds((rp * 2) * _B + cc * 2048, 2048)])
            pltpu.sync_copy(
                ibuf.at[1],
                idx_lin.at[pl.ds((rp * 2 + 1) * _B + cc * 2048, 2048)])
            return carry

        nq = (200 - w + _NW - 1) // _NW
        lax.fori_loop(0, nq, lambda q, c: ichunk(q * _NW + w, c), 0,
                      unroll=False)

    return k(tabT, idxT, tail)


def _gather_fmt(idx_lin2, tab_lin):
    mesh = plsc.VectorSubcoreMesh(core_axis_name="c", subcore_axis_name="s")
    n_blocks = 200

    @functools.partial(
        pl.kernel,
        mesh=mesh,
        out_type=jax.ShapeDtypeStruct((_H, 4, _VBLK, 8, _VBLK), jnp.float32),
        scratch_types=[
            pltpu.VMEM((_H, 512), jnp.int32),
            pltpu.VMEM((_VBLK, _D), jnp.float32),
            pltpu.VMEM((_VBLK, _D), jnp.float32),
            pltpu.VMEM((4, 8, _VBLK), jnp.float32),
            pltpu.VMEM((4, 8, _VBLK), jnp.float32),
            pltpu.SemaphoreType.DMA,
            pltpu.SemaphoreType.DMA,
            pltpu.SemaphoreType.DMA,
            pltpu.SemaphoreType.DMA,
        ],
        compiler_params=pltpu.CompilerParams(
            use_tc_tiling_on_sc=False, needs_layout_passes=False),
    )
    def k(idx_hbm, tab_hbm, out_hbm, idx_all, g0, g1, t0, t1,
          sg0, sg1, st0, st1):
        i16 = lax.iota(jnp.int32, 16)
        w = _wid()
        gbuf = (g0, g1)
        tbuf = (t0, t1)
        sg = (sg0, sg1)
        st = (st0, st1)

        pltpu.sync_copy(idx_hbm.at[:, pl.ds(w * 512, 512)], idx_all)

        def gather_in(t, b):
            pltpu.async_copy(
                tab_hbm.at[idx_all.at[t // 4, pl.ds((t % 4) * _VBLK, _VBLK)]],
                gbuf[b], sg[b])

        def store_out(t, b):
            h = t // 4
            c = (w * 4) + (t % 4)
            pltpu.async_copy(tbuf[b], out_hbm.at[h, :, c], st[b])

        def wait_g(b):
            pltpu.make_async_copy(
                out_hbm.at[0, :, 0], gbuf[b], sg[b]).wait()

        def wait_s(b):
            pltpu.make_async_copy(
                out_hbm.at[0, :, 0], tbuf[b], st[b]).wait()

        def transpose(b):
            for d in range(_D):
                dcol = jnp.full((16,), d, jnp.int32)
                for kk in range(8):
                    vec = plsc.load_gather(gbuf[b], [i16 + (kk * 16), dcol])
                    tbuf[b][d // 8, d % 8, pl.ds(kk * 16, 16)] = vec

        gather_in(0, 0)
        wait_g(0)
        gather_in(1, 1)
        transpose(0)
        store_out(0, 0)
        wait_g(1)
        gather_in(2, 0)
        transpose(1)
        store_out(1, 1)

        def body(p, carry):
            c0 = p * 2
            wait_g(0)
            wait_s(1)
            gather_in(c0 + 1, 1)
            transpose(0)
            store_out(c0, 0)
            wait_g(1)
            wait_s(0)
            gather_in(c0 + 2, 0)
            transpose(1)
            store_out(c0 + 1, 1)
            return carry

        lax.fori_loop(1, n_blocks // 2 - 1, body, 0)

        cl = n_blocks - 2
        wait_g(0)
        wait_s(1)
        gather_in(cl + 1, 1)
        transpose(0)
        store_out(cl, 0)
        wait_g(1)
        transpose(1)
        store_out(cl + 1, 1)
        wait_s(0)
        wait_s(1)

    return k(idx_lin2, tab_lin)


def kernel(sentences_idx, table):
    tabT = table.T
    idxT = sentences_idx.astype(jnp.int32).T
    tail = table[_NFULL * _VBLK:]
    tab_flat, idx_flat = _detile(tabT, idxT, tail)
    tab_lin = tab_flat.reshape(_V, _D)
    idx_lin2 = idx_flat.reshape(_H, _B)
    out5 = _gather_fmt(idx_lin2, tab_lin)
    return out5.transpose(2, 4, 0, 1, 3).reshape(_B, _H, _D)


# speedup vs baseline: 1.5405x; 1.3173x over previous
import functools

import jax
import jax.numpy as jnp
from jax import lax
from jax.experimental import pallas as pl
from jax.experimental.pallas import tpu as pltpu
from jax.experimental.pallas import tpu_sc as plsc

_V = 1000000
_D = 32
_B = 16384
_H = 50
_NW = 32
_VBLK = 128
_NFULL = _V // _VBLK
_TAIL = _V - _NFULL * _VBLK
_MAIN = 7808
_PER_W = _MAIN // _NW


def _wid():
    return lax.axis_index("s") * 2 + lax.axis_index("c")


def _detile(tabT, idxT, tail):
    mesh = plsc.VectorSubcoreMesh(core_axis_name="c", subcore_axis_name="s")

    @functools.partial(
        pl.kernel,
        mesh=mesh,
        out_type=(
            jax.ShapeDtypeStruct((_V * _D,), jnp.float32),
            jax.ShapeDtypeStruct((_H * _B,), jnp.int32),
        ),
        scratch_types=[
            pltpu.VMEM((_D, _VBLK), jnp.float32),
            pltpu.VMEM((_D, _VBLK), jnp.float32),
            pltpu.VMEM((_VBLK * _D,), jnp.float32),
            pltpu.VMEM((_VBLK * _D,), jnp.float32),
            pltpu.VMEM((2, 2048), jnp.int32),
            pltpu.VMEM((_TAIL, _D), jnp.float32),
            pltpu.VMEM((_TAIL * _D,), jnp.float32),
            pltpu.SemaphoreType.DMA,
            pltpu.SemaphoreType.DMA,
            pltpu.SemaphoreType.DMA,
            pltpu.SemaphoreType.DMA,
        ],
        compiler_params=pltpu.CompilerParams(needs_layout_passes=False),
    )
    def k(tabT_hbm, idxT_hbm, tail_hbm, tab_lin, idx_lin,
          g0, g1, t0, t1, ibuf, gt, tt, sg0, sg1, st0, st1):
        i16 = lax.iota(jnp.int32, 16)
        w = _wid()
        gbuf = (g0, g1)
        tbuf = (t0, t1)
        sg = (sg0, sg1)
        st = (st0, st1)

        def blk_of(i):
            return w * _PER_W + i

        def gather_in(i, b):
            pltpu.async_copy(
                tabT_hbm.at[:, pl.ds(blk_of(i) * _VBLK, _VBLK)], gbuf[b], sg[b])

        def store_out(i, b):
            pltpu.async_copy(
                tbuf[b], tab_lin.at[pl.ds(blk_of(i) * _VBLK * _D, _VBLK * _D)],
                st[b])

        def wait_g(b):
            pltpu.make_async_copy(
                tabT_hbm.at[:, pl.ds(0, _VBLK)], gbuf[b], sg[b]).wait()

        def wait_s(b):
            pltpu.make_async_copy(
                tab_lin.at[pl.ds(0, _VBLK * _D)], tbuf[b], st[b]).wait()

        def transpose(b):
            base = i16 * _D
            for d in range(_D):
                for kk in range(8):
                    vec = gbuf[b][d, pl.ds(kk * 16, 16)]
                    plsc.store_scatter(
                        tbuf[b], [base + (kk * 16 * _D + d)], vec)

        gather_in(0, 0)
        wait_g(0)
        gather_in(1, 1)
        transpose(0)
        store_out(0, 0)
        wait_g(1)
        gather_in(2, 0)
        transpose(1)
        store_out(1, 1)

        def body(p, carry):
            c0 = p * 2
            wait_g(0)
            wait_s(1)
            gather_in(c0 + 1, 1)
            transpose(0)
            store_out(c0, 0)
            wait_g(1)
            wait_s(0)
            gather_in(c0 + 2, 0)
            transpose(1)
            store_out(c0 + 1, 1)
            return carry

        lax.fori_loop(1, _PER_W // 2 - 1, body, 0)

        cl = _PER_W - 2
        wait_g(0)
        wait_s(1)
        gather_in(cl + 1, 1)
        transpose(0)
        store_out(cl, 0)
        wait_g(1)
        transpose(1)
        store_out(cl + 1, 1)
        wait_s(0)
        wait_s(1)

        @pl.when(w >= 28)
        def _extra():
            blk = _MAIN + (w - 28)
            pltpu.async_copy(
                tabT_hbm.at[:, pl.ds(blk * _VBLK, _VBLK)], g0, sg0)
            wait_g(0)
            transpose(0)
            pltpu.async_copy(
                t0, tab_lin.at[pl.ds(blk * _VBLK * _D, _VBLK * _D)], st0)
            wait_s(0)

        @pl.when(w == 27)
        def _tail():
            pltpu.sync_copy(tail_hbm, gt)
            for r in range(_TAIL):
                for kk in range(2):
                    tt[pl.ds(r * _D + kk * 16, 16)] = gt[r, pl.ds(kk * 16, 16)]
            pltpu.async_copy(
                tt, tab_lin.at[pl.ds(_NFULL * _VBLK * _D, _TAIL * _D)], st0)
            pltpu.make_async_copy(tab_lin.at[pl.ds(0, _TAIL * _D)], tt,
                                  st0).wait()

        def ichunk(q, carry):
            rp = q // 8
            cc = q % 8
            pltpu.sync_copy(
                idxT_hbm.at[pl.ds(rp * 2, 2), pl.ds(cc * 2048, 2048)], ibuf)
            pltpu.sync_copy(
                ibuf.at[0],
                idx_lin.at[pl.ds((rp * 2) * _B + cc * 2048, 2048)])
            pltpu.sync_copy(
                ibuf.at[1],
                idx_lin.at[pl.ds((rp * 2 + 1) * _B + cc * 2048, 2048)])
            return carry

        nq = (200 - w + _NW - 1) // _NW
        lax.fori_loop(0, nq, lambda q, c: ichunk(q * _NW + w, c), 0,
                      unroll=False)

    return k(tabT, idxT, tail)


def _gather_fmt(idx_lin2, tab_lin):
    mesh = plsc.VectorSubcoreMesh(core_axis_name="c", subcore_axis_name="s")
    n_blocks = 200

    @functools.partial(
        pl.kernel,
        mesh=mesh,
        out_type=jax.ShapeDtypeStruct((_H * 4 * _VBLK * 8 * _VBLK,),
                                      jnp.float32),
        scratch_types=[
            pltpu.VMEM((_H, 512), jnp.int32),
            pltpu.VMEM((_VBLK, _D), jnp.float32),
            pltpu.VMEM((_VBLK, _D), jnp.float32),
            pltpu.VMEM((_D * _VBLK,), jnp.float32),
            pltpu.VMEM((_D * _VBLK,), jnp.float32),
            pltpu.SemaphoreType.DMA,
            pltpu.SemaphoreType.DMA,
            pltpu.SemaphoreType.DMA,
            pltpu.SemaphoreType.DMA,
        ],
        compiler_params=pltpu.CompilerParams(
            use_tc_tiling_on_sc=False, needs_layout_passes=False),
    )
    def k(idx_hbm, tab_hbm, out_hbm, idx_all, g0, g1, t0, t1,
          sg0, sg1, st0, st1):
        i16 = lax.iota(jnp.int32, 16)
        w = _wid()
        gbuf = (g0, g1)
        tbuf = (t0, t1)
        sg = (sg0, sg1)
        st = (st0, st1)

        pltpu.sync_copy(idx_hbm.at[:, pl.ds(w * 512, 512)], idx_all)

        def gather_in(t, b):
            pltpu.async_copy(
                tab_hbm.at[idx_all.at[t // 4, pl.ds((t % 4) * _VBLK, _VBLK)]],
                gbuf[b], sg[b])

        def store_out(t, b):
            h = t // 4
            c = (w * 4) + (t % 4)
            for a in range(4):
                pltpu.async_copy(
                    tbuf[b].at[pl.ds(a * 1024, 1024)],
                    out_hbm.at[pl.ds(((h * 4 + a) * _VBLK + c) * 1024, 1024)],
                    st[b])

        def wait_g(b):
            pltpu.make_async_copy(
                out_hbm.at[pl.ds(0, _VBLK * _D)], gbuf[b], sg[b]).wait()

        def wait_s(b):
            for _ in range(4):
                pltpu.make_async_copy(
                    out_hbm.at[pl.ds(0, 1024)], tbuf[b].at[pl.ds(0, 1024)],
                    st[b]).wait()

        def transpose(b):
            base = i16 * _VBLK
            for l in range(_VBLK):
                for kk in range(2):
                    vec = gbuf[b][l, pl.ds(kk * 16, 16)]
                    plsc.store_scatter(
                        tbuf[b], [base + (kk * 16 * _VBLK + l)], vec)

        gather_in(0, 0)
        wait_g(0)
        gather_in(1, 1)
        transpose(0)
        store_out(0, 0)
        wait_g(1)
        gather_in(2, 0)
        transpose(1)
        store_out(1, 1)

        def body(p, carry):
            c0 = p * 2
            wait_g(0)
            wait_s(1)
            gather_in(c0 + 1, 1)
            transpose(0)
            store_out(c0, 0)
            wait_g(1)
            wait_s(0)
            gather_in(c0 + 2, 0)
            transpose(1)
            store_out(c0 + 1, 1)
            return carry

        lax.fori_loop(1, n_blocks // 2 - 1, body, 0)

        cl = n_blocks - 2
        wait_g(0)
        wait_s(1)
        gather_in(cl + 1, 1)
        transpose(0)
        store_out(cl, 0)
        wait_g(1)
        transpose(1)
        store_out(cl + 1, 1)
        wait_s(0)
        wait_s(1)

    return k(idx_lin2, tab_lin)


def kernel(sentences_idx, table):
    tabT = table.T
    idxT = sentences_idx.astype(jnp.int32).T
    tail = table[_NFULL * _VBLK:]
    tab_flat, idx_flat = _detile(tabT, idxT, tail)
    tab_lin = tab_flat.reshape(_V, _D)
    idx_lin2 = idx_flat.reshape(_H, _B)
    out_flat = _gather_fmt(idx_lin2, tab_lin)
    out5 = out_flat.reshape(_H, 4, _VBLK, 8, _VBLK)
    return out5.transpose(2, 4, 0, 1, 3).reshape(_B, _H, _D)


# speedup vs baseline: 1.7513x; 1.1368x over previous
import functools

import jax
import jax.numpy as jnp
from jax import lax
from jax.experimental import pallas as pl
from jax.experimental.pallas import tpu as pltpu
from jax.experimental.pallas import tpu_sc as plsc

_V = 1000000
_D = 32
_B = 16384
_H = 50
_NW = 32
_VBLK = 128
_NFULL = _V // _VBLK
_TAIL = _V - _NFULL * _VBLK
_MAIN = 7808
_PER_W = _MAIN // _NW


def _wid():
    return lax.axis_index("s") * 2 + lax.axis_index("c")


def _detile(tabT, idxT, tail):
    mesh = plsc.VectorSubcoreMesh(core_axis_name="c", subcore_axis_name="s")

    @functools.partial(
        pl.kernel,
        mesh=mesh,
        out_type=(
            jax.ShapeDtypeStruct((_V * _D,), jnp.float32),
            jax.ShapeDtypeStruct((_H * _B,), jnp.int32),
        ),
        scratch_types=[
            pltpu.VMEM((_D, _VBLK), jnp.float32),
            pltpu.VMEM((_D, _VBLK), jnp.float32),
            pltpu.VMEM((_VBLK * _D,), jnp.float32),
            pltpu.VMEM((_VBLK * _D,), jnp.float32),
            pltpu.VMEM((2, 2048), jnp.int32),
            pltpu.VMEM((_TAIL, _D), jnp.float32),
            pltpu.VMEM((_TAIL * _D,), jnp.float32),
            pltpu.SemaphoreType.DMA,
            pltpu.SemaphoreType.DMA,
            pltpu.SemaphoreType.DMA,
            pltpu.SemaphoreType.DMA,
        ],
        compiler_params=pltpu.CompilerParams(needs_layout_passes=False),
    )
    def k(tabT_hbm, idxT_hbm, tail_hbm, tab_lin, idx_lin,
          g0, g1, t0, t1, ibuf, gt, tt, sg0, sg1, st0, st1):
        i16 = lax.iota(jnp.int32, 16)
        w = _wid()
        gbuf = (g0, g1)
        tbuf = (t0, t1)
        sg = (sg0, sg1)
        st = (st0, st1)

        def blk_of(i):
            return w * _PER_W + i

        def gather_in(i, b):
            pltpu.async_copy(
                tabT_hbm.at[:, pl.ds(blk_of(i) * _VBLK, _VBLK)], gbuf[b], sg[b])

        def store_out(i, b):
            pltpu.async_copy(
                tbuf[b], tab_lin.at[pl.ds(blk_of(i) * _VBLK * _D, _VBLK * _D)],
                st[b])

        def wait_g(b):
            pltpu.make_async_copy(
                tabT_hbm.at[:, pl.ds(0, _VBLK)], gbuf[b], sg[b]).wait()

        def wait_s(b):
            pltpu.make_async_copy(
                tab_lin.at[pl.ds(0, _VBLK * _D)], tbuf[b], st[b]).wait()

        def transpose(b):
            base = i16 * _D
            for d in range(_D):
                for kk in range(8):
                    vec = gbuf[b][d, pl.ds(kk * 16, 16)]
                    plsc.store_scatter(
                        tbuf[b], [base + (kk * 16 * _D + d)], vec)

        gather_in(0, 0)
        wait_g(0)
        gather_in(1, 1)
        transpose(0)
        store_out(0, 0)
        wait_g(1)
        gather_in(2, 0)
        transpose(1)
        store_out(1, 1)

        def body(p, carry):
            c0 = p * 2
            wait_g(0)
            wait_s(1)
            gather_in(c0 + 1, 1)
            transpose(0)
            store_out(c0, 0)
            wait_g(1)
            wait_s(0)
            gather_in(c0 + 2, 0)
            transpose(1)
            store_out(c0 + 1, 1)
            return carry

        lax.fori_loop(1, _PER_W // 2 - 1, body, 0)

        cl = _PER_W - 2
        wait_g(0)
        wait_s(1)
        gather_in(cl + 1, 1)
        transpose(0)
        store_out(cl, 0)
        wait_g(1)
        transpose(1)
        store_out(cl + 1, 1)
        wait_s(0)
        wait_s(1)

        @pl.when(w >= 28)
        def _extra():
            blk = _MAIN + (w - 28)
            pltpu.async_copy(
                tabT_hbm.at[:, pl.ds(blk * _VBLK, _VBLK)], g0, sg0)
            wait_g(0)
            transpose(0)
            pltpu.async_copy(
                t0, tab_lin.at[pl.ds(blk * _VBLK * _D, _VBLK * _D)], st0)
            wait_s(0)

        @pl.when(w == 27)
        def _tail():
            pltpu.sync_copy(tail_hbm, gt)
            for r in range(_TAIL):
                for kk in range(2):
                    tt[pl.ds(r * _D + kk * 16, 16)] = gt[r, pl.ds(kk * 16, 16)]
            pltpu.async_copy(
                tt, tab_lin.at[pl.ds(_NFULL * _VBLK * _D, _TAIL * _D)], st0)
            pltpu.make_async_copy(tab_lin.at[pl.ds(0, _TAIL * _D)], tt,
                                  st0).wait()

        def ichunk(q, carry):
            rp = q // 8
            cc = q % 8
            pltpu.sync_copy(
                idxT_hbm.at[pl.ds(rp * 2, 2), pl.ds(cc * 2048, 2048)], ibuf)
            pltpu.sync_copy(
                ibuf.at[0],
                idx_lin.at[pl.ds((rp * 2) * _B + cc * 2048, 2048)])
            pltpu.sync_copy(
                ibuf.at[1],
                idx_lin.at[pl.ds((rp * 2 + 1) * _B + cc * 2048, 2048)])
            return carry

        nq = (200 - w + _NW - 1) // _NW
        lax.fori_loop(0, nq, lambda q, c: ichunk(q * _NW + w, c), 0,
                      unroll=False)

    return k(tabT, idxT, tail)


def _gather_fmt(idx_lin1, tab_lin):
    mesh = plsc.VectorSubcoreMesh(core_axis_name="c", subcore_axis_name="s")
    n_blocks = 200

    @functools.partial(
        pl.kernel,
        mesh=mesh,
        out_type=jax.ShapeDtypeStruct((_H * 4 * _VBLK * 8 * _VBLK,),
                                      jnp.float32),
        scratch_types=[
            pltpu.VMEM((_H, 512), jnp.int32),
            pltpu.VMEM((_VBLK, _D), jnp.float32),
            pltpu.VMEM((_VBLK, _D), jnp.float32),
            pltpu.VMEM((_D * _VBLK,), jnp.float32),
            pltpu.VMEM((_D * _VBLK,), jnp.float32),
            pltpu.SemaphoreType.DMA,
            pltpu.SemaphoreType.DMA,
            pltpu.SemaphoreType.DMA,
            pltpu.SemaphoreType.DMA,
        ],
        compiler_params=pltpu.CompilerParams(
            use_tc_tiling_on_sc=False, needs_layout_passes=False),
    )
    def k(idx_hbm, tab_hbm, out_hbm, idx_all, g0, g1, t0, t1,
          sg0, sg1, st0, st1):
        i16 = lax.iota(jnp.int32, 16)
        w = _wid()
        gbuf = (g0, g1)
        tbuf = (t0, t1)
        sg = (sg0, sg1)
        st = (st0, st1)

        for hh in range(_H):
            pltpu.sync_copy(idx_hbm.at[pl.ds(hh * _B + w * 512, 512)],
                            idx_all.at[hh])

        def gather_in(t, b):
            pltpu.async_copy(
                tab_hbm.at[idx_all.at[t // 4, pl.ds((t % 4) * _VBLK, _VBLK)]],
                gbuf[b], sg[b])

        def store_out(t, b):
            h = t // 4
            c = (w * 4) + (t % 4)
            for a in range(4):
                pltpu.async_copy(
                    tbuf[b].at[pl.ds(a * 1024, 1024)],
                    out_hbm.at[pl.ds(((h * 4 + a) * _VBLK + c) * 1024, 1024)],
                    st[b])

        def wait_g(b):
            pltpu.make_async_copy(
                out_hbm.at[pl.ds(0, _VBLK * _D)], gbuf[b], sg[b]).wait()

        def wait_s(b):
            for _ in range(4):
                pltpu.make_async_copy(
                    out_hbm.at[pl.ds(0, 1024)], tbuf[b].at[pl.ds(0, 1024)],
                    st[b]).wait()

        def transpose(b):
            base = i16 * _VBLK
            for l in range(_VBLK):
                for kk in range(2):
                    vec = gbuf[b][l, pl.ds(kk * 16, 16)]
                    plsc.store_scatter(
                        tbuf[b], [base + (kk * 16 * _VBLK + l)], vec)

        gather_in(0, 0)
        wait_g(0)
        gather_in(1, 1)
        transpose(0)
        store_out(0, 0)
        wait_g(1)
        gather_in(2, 0)
        transpose(1)
        store_out(1, 1)

        def body(p, carry):
            c0 = p * 2
            wait_g(0)
            wait_s(1)
            gather_in(c0 + 1, 1)
            transpose(0)
            store_out(c0, 0)
            wait_g(1)
            wait_s(0)
            gather_in(c0 + 2, 0)
            transpose(1)
            store_out(c0 + 1, 1)
            return carry

        lax.fori_loop(1, n_blocks // 2 - 1, body, 0)

        cl = n_blocks - 2
        wait_g(0)
        wait_s(1)
        gather_in(cl + 1, 1)
        transpose(0)
        store_out(cl, 0)
        wait_g(1)
        transpose(1)
        store_out(cl + 1, 1)
        wait_s(0)
        wait_s(1)

    return k(idx_lin1, tab_lin)


def kernel(sentences_idx, table):
    idx_lin1 = sentences_idx.astype(jnp.int32).T.reshape(_H * _B)
    out_flat = _gather_fmt(idx_lin1, table)
    out5 = out_flat.reshape(_H, 4, _VBLK, 8, _VBLK)
    return out5.transpose(2, 4, 0, 1, 3).reshape(_B, _H, _D)


# speedup vs baseline: 1.7570x; 1.0033x over previous
import functools

import jax
import jax.numpy as jnp
from jax import lax
from jax.experimental import pallas as pl
from jax.experimental.pallas import tpu as pltpu
from jax.experimental.pallas import tpu_sc as plsc

_V = 1000000
_D = 32
_B = 16384
_H = 50
_NW = 32
_VBLK = 128
_NFULL = _V // _VBLK
_TAIL = _V - _NFULL * _VBLK
_MAIN = 7808
_PER_W = _MAIN // _NW


def _wid():
    return lax.axis_index("s") * 2 + lax.axis_index("c")


def _detile(tabT, idxT, tail):
    mesh = plsc.VectorSubcoreMesh(core_axis_name="c", subcore_axis_name="s")

    @functools.partial(
        pl.kernel,
        mesh=mesh,
        out_type=(
            jax.ShapeDtypeStruct((_V * _D,), jnp.float32),
            jax.ShapeDtypeStruct((_H * _B,), jnp.int32),
        ),
        scratch_types=[
            pltpu.VMEM((_D, _VBLK), jnp.float32),
            pltpu.VMEM((_D, _VBLK), jnp.float32),
            pltpu.VMEM((_VBLK * _D,), jnp.float32),
            pltpu.VMEM((_VBLK * _D,), jnp.float32),
            pltpu.VMEM((2, 2048), jnp.int32),
            pltpu.VMEM((_TAIL, _D), jnp.float32),
            pltpu.VMEM((_TAIL * _D,), jnp.float32),
            pltpu.SemaphoreType.DMA,
            pltpu.SemaphoreType.DMA,
            pltpu.SemaphoreType.DMA,
            pltpu.SemaphoreType.DMA,
        ],
        compiler_params=pltpu.CompilerParams(needs_layout_passes=False),
    )
    def k(tabT_hbm, idxT_hbm, tail_hbm, tab_lin, idx_lin,
          g0, g1, t0, t1, ibuf, gt, tt, sg0, sg1, st0, st1):
        i16 = lax.iota(jnp.int32, 16)
        w = _wid()
        gbuf = (g0, g1)
        tbuf = (t0, t1)
        sg = (sg0, sg1)
        st = (st0, st1)

        def blk_of(i):
            return w * _PER_W + i

        def gather_in(i, b):
            pltpu.async_copy(
                tabT_hbm.at[:, pl.ds(blk_of(i) * _VBLK, _VBLK)], gbuf[b], sg[b])

        def store_out(i, b):
            pltpu.async_copy(
                tbuf[b], tab_lin.at[pl.ds(blk_of(i) * _VBLK * _D, _VBLK * _D)],
                st[b])

        def wait_g(b):
            pltpu.make_async_copy(
                tabT_hbm.at[:, pl.ds(0, _VBLK)], gbuf[b], sg[b]).wait()

        def wait_s(b):
            pltpu.make_async_copy(
                tab_lin.at[pl.ds(0, _VBLK * _D)], tbuf[b], st[b]).wait()

        def transpose(b):
            base = i16 * _D
            for d in range(_D):
                for kk in range(8):
                    vec = gbuf[b][d, pl.ds(kk * 16, 16)]
                    plsc.store_scatter(
                        tbuf[b], [base + (kk * 16 * _D + d)], vec)

        gather_in(0, 0)
        wait_g(0)
        gather_in(1, 1)
        transpose(0)
        store_out(0, 0)
        wait_g(1)
        gather_in(2, 0)
        transpose(1)
        store_out(1, 1)

        def body(p, carry):
            c0 = p * 2
            wait_g(0)
            wait_s(1)
            gather_in(c0 + 1, 1)
            transpose(0)
            store_out(c0, 0)
            wait_g(1)
            wait_s(0)
            gather_in(c0 + 2, 0)
            transpose(1)
            store_out(c0 + 1, 1)
            return carry

        lax.fori_loop(1, _PER_W // 2 - 1, body, 0)

        cl = _PER_W - 2
        wait_g(0)
        wait_s(1)
        gather_in(cl + 1, 1)
        transpose(0)
        store_out(cl, 0)
        wait_g(1)
        transpose(1)
        store_out(cl + 1, 1)
        wait_s(0)
        wait_s(1)

        @pl.when(w >= 28)
        def _extra():
            blk = _MAIN + (w - 28)
            pltpu.async_copy(
                tabT_hbm.at[:, pl.ds(blk * _VBLK, _VBLK)], g0, sg0)
            wait_g(0)
            transpose(0)
            pltpu.async_copy(
                t0, tab_lin.at[pl.ds(blk * _VBLK * _D, _VBLK * _D)], st0)
            wait_s(0)

        @pl.when(w == 27)
        def _tail():
            pltpu.sync_copy(tail_hbm, gt)
            for r in range(_TAIL):
                for kk in range(2):
                    tt[pl.ds(r * _D + kk * 16, 16)] = gt[r, pl.ds(kk * 16, 16)]
            pltpu.async_copy(
                tt, tab_lin.at[pl.ds(_NFULL * _VBLK * _D, _TAIL * _D)], st0)
            pltpu.make_async_copy(tab_lin.at[pl.ds(0, _TAIL * _D)], tt,
                                  st0).wait()

        def ichunk(q, carry):
            rp = q // 8
            cc = q % 8
            pltpu.sync_copy(
                idxT_hbm.at[pl.ds(rp * 2, 2), pl.ds(cc * 2048, 2048)], ibuf)
            pltpu.sync_copy(
                ibuf.at[0],
                idx_lin.at[pl.ds((rp * 2) * _B + cc * 2048, 2048)])
            pltpu.sync_copy(
                ibuf.at[1],
                idx_lin.at[pl.ds((rp * 2 + 1) * _B + cc * 2048, 2048)])
            return carry

        nq = (200 - w + _NW - 1) // _NW
        lax.fori_loop(0, nq, lambda q, c: ichunk(q * _NW + w, c), 0,
                      unroll=False)

    return k(tabT, idxT, tail)


def _gather_fmt(idx2d, tab_lin):
    mesh = plsc.VectorSubcoreMesh(core_axis_name="c", subcore_axis_name="s")
    n_blocks = 200

    @functools.partial(
        pl.kernel,
        mesh=mesh,
        out_type=jax.ShapeDtypeStruct((_H * 4 * _VBLK * 8 * _VBLK,),
                                      jnp.float32),
        scratch_types=[
            pltpu.VMEM((_H, 512), jnp.int32),
            pltpu.VMEM((_VBLK, _D), jnp.float32),
            pltpu.VMEM((_VBLK, _D), jnp.float32),
            pltpu.VMEM((_D * _VBLK,), jnp.float32),
            pltpu.VMEM((_D * _VBLK,), jnp.float32),
            pltpu.SemaphoreType.DMA,
            pltpu.SemaphoreType.DMA,
            pltpu.SemaphoreType.DMA,
            pltpu.SemaphoreType.DMA,
        ],
        compiler_params=pltpu.CompilerParams(
            use_tc_tiling_on_sc=False, needs_layout_passes=False),
    )
    def k(idx_hbm, tab_hbm, out_hbm, idx_all, g0, g1, t0, t1,
          sg0, sg1, st0, st1):
        i16 = lax.iota(jnp.int32, 16)
        w = _wid()
        gbuf = (g0, g1)
        tbuf = (t0, t1)
        sg = (sg0, sg1)
        st = (st0, st1)

        for hh in range(_H):
            pltpu.sync_copy(idx_hbm.at[hh, pl.ds(w * 512, 512)],
                            idx_all.at[hh])

        def gather_in(t, b):
            pltpu.async_copy(
                tab_hbm.at[idx_all.at[t // 4, pl.ds((t % 4) * _VBLK, _VBLK)]],
                gbuf[b], sg[b])

        def store_out(t, b):
            h = t // 4
            c = (w * 4) + (t % 4)
            for a in range(4):
                pltpu.async_copy(
                    tbuf[b].at[pl.ds(a * 1024, 1024)],
                    out_hbm.at[pl.ds(((h * 4 + a) * _VBLK + c) * 1024, 1024)],
                    st[b])

        def wait_g(b):
            pltpu.make_async_copy(
                out_hbm.at[pl.ds(0, _VBLK * _D)], gbuf[b], sg[b]).wait()

        def wait_s(b):
            for _ in range(4):
                pltpu.make_async_copy(
                    out_hbm.at[pl.ds(0, 1024)], tbuf[b].at[pl.ds(0, 1024)],
                    st[b]).wait()

        def transpose(b):
            base = i16 * _VBLK
            for l in range(_VBLK):
                for kk in range(2):
                    vec = gbuf[b][l, pl.ds(kk * 16, 16)]
                    plsc.store_scatter(
                        tbuf[b], [base + (kk * 16 * _VBLK + l)], vec)

        gather_in(0, 0)
        wait_g(0)
        gather_in(1, 1)
        transpose(0)
        store_out(0, 0)
        wait_g(1)
        gather_in(2, 0)
        transpose(1)
        store_out(1, 1)

        def body(p, carry):
            c0 = p * 2
            wait_g(0)
            wait_s(1)
            gather_in(c0 + 1, 1)
            transpose(0)
            store_out(c0, 0)
            wait_g(1)
            wait_s(0)
            gather_in(c0 + 2, 0)
            transpose(1)
            store_out(c0 + 1, 1)
            return carry

        lax.fori_loop(1, n_blocks // 2 - 1, body, 0)

        cl = n_blocks - 2
        wait_g(0)
        wait_s(1)
        gather_in(cl + 1, 1)
        transpose(0)
        store_out(cl, 0)
        wait_g(1)
        transpose(1)
        store_out(cl + 1, 1)
        wait_s(0)
        wait_s(1)

    return k(idx2d, tab_lin)


def kernel(sentences_idx, table):
    idxT = sentences_idx.astype(jnp.int32).T
    out_flat = _gather_fmt(idxT, table)
    out5 = out_flat.reshape(_H, 4, _VBLK, 8, _VBLK)
    return out5.transpose(2, 4, 0, 1, 3).reshape(_B, _H, _D)
